# baseline (speedup 1.0000x reference)
import functools
import math

import jax
import jax.numpy as jnp
from jax import lax
from jax.experimental import pallas as pl
from jax.experimental.pallas import tpu as pltpu
from jax.experimental.pallas import tpu_sc as plsc

_SMOOTHING = 0.1
_VOCAB = 32000
_N_TOKENS = 2048
_CONF = 1.0 - _SMOOTHING
_SVAL = _SMOOTHING / float(_VOCAB - 2)
_C_HIT = (_VOCAB - 2) * _SVAL * math.log(_SVAL) + _CONF * math.log(_CONF)
_C_IGN = (_VOCAB - 1) * _SVAL * math.log(_SVAL) + _CONF * math.log(_CONF)

_BC = 3200
_NBLK = _VOCAB // _BC

_LN = 16
_NWORKERS = 32
_TOK_PER_W = _N_TOKENS // _NWORKERS
_NCHUNK = _TOK_PER_W // _LN


def _sc_body(table_hbm, tgt_hbm, s_hbm, z_hbm, out_hbm, tgt_v, s_v, z_v,
             lane_v, trows_v, out_v, semt):
    wid = lax.axis_index("s") * 2 + lax.axis_index("c")
    base = wid * _TOK_PER_W
    pltpu.sync_copy(tgt_hbm.at[pl.ds(base, _TOK_PER_W)], tgt_v)
    pltpu.sync_copy(s_hbm.at[pl.ds(base, _TOK_PER_W)], s_v)
    pltpu.sync_copy(z_hbm.at[pl.ds(base, _TOK_PER_W)], z_v)
    for k in range(_NCHUNK):
        t16 = tgt_v[pl.ds(k * _LN, _LN)]
        lane_v[pl.ds(k * _LN, _LN)] = t16 & 127
    tcopies = []
    lane_iota = lax.iota(jnp.int32, _LN)
    for j in range(_TOK_PER_W):
        t16 = tgt_v[pl.ds((j // _LN) * _LN, _LN)]
        tj = jnp.sum(jnp.where(lane_iota == (j % _LN), t16, 0))
        c128 = pl.multiple_of(tj - (tj & 127), 128)
        r8 = pl.multiple_of(base + (j // 8) * 8, 8)
        cp = pltpu.make_async_copy(
            table_hbm.at[pl.ds(r8, 8), pl.ds(c128, 128)],
            trows_v.at[j], semt)
        cp.start()
        tcopies.append(cp)
    for cp in tcopies:
        cp.wait()
    for k in range(_NCHUNK):
        loc = k * _LN + lane_iota
        sub = loc & 7
        ln = lane_v[pl.ds(k * _LN, _LN)]
        tg = tgt_v[pl.ds(k * _LN, _LN)]
        sv = s_v[pl.ds(k * _LN, _LN)]
        z = z_v[pl.ds(k * _LN, _LN)]
        t = plsc.load_gather(trows_v, [loc, sub, ln])
        hit_val = _C_HIT + _SVAL * z + (_SVAL - _CONF) * t
        ign_val = _C_IGN + (_SVAL - _CONF) * z
        out_v[pl.ds(k * _LN, _LN)] = jnp.where(
            tg == 0, ign_val, hit_val) - _SVAL * sv
    pltpu.sync_copy(out_v, out_hbm.at[pl.ds(base, _TOK_PER_W)])


def _sc_gather_combine(table, tgt, s1d, z1d):
    mesh = plsc.VectorSubcoreMesh(core_axis_name="c", subcore_axis_name="s")
    f = pl.kernel(
        _sc_body,
        out_type=jax.ShapeDtypeStruct((_N_TOKENS,), jnp.float32),
        mesh=mesh,
        scratch_types=[
            pltpu.VMEM((_TOK_PER_W,), jnp.int32),
            pltpu.VMEM((_TOK_PER_W,), jnp.float32),
            pltpu.VMEM((_TOK_PER_W,), jnp.float32),
            pltpu.VMEM((_TOK_PER_W,), jnp.int32),
            pltpu.VMEM((_TOK_PER_W, 8, 128), jnp.float32),
            pltpu.VMEM((_TOK_PER_W,), jnp.float32),
            pltpu.SemaphoreType.DMA,
        ],
        compiler_params=pltpu.CompilerParams(needs_layout_passes=False),
    )
    return f(table, tgt, s1d, z1d)


_BR = 128
_NRB = _N_TOKENS // _BR


def _tc_rowsum_body(x_ref, s_ref, z_ref):
    s_ref[...] = jnp.sum(x_ref[...], axis=1, keepdims=True)
    z_ref[...] = x_ref[:, 0:1]


def _tc_rowsum(x, interpret=False):
    return pl.pallas_call(
        _tc_rowsum_body,
        grid=(_NRB,),
        in_specs=[pl.BlockSpec((_BR, _VOCAB), lambda j: (j, 0))],
        out_specs=[pl.BlockSpec((_BR, 1), lambda j: (j, 0)),
                   pl.BlockSpec((_BR, 1), lambda j: (j, 0))],
        out_shape=[jax.ShapeDtypeStruct((_N_TOKENS, 1), jnp.float32),
                   jax.ShapeDtypeStruct((_N_TOKENS, 1), jnp.float32)],
        interpret=interpret,
    )(x)


def kernel(model_prob, target):
    tgt = target.astype(jnp.int32)
    s2d, z2d = _tc_rowsum(model_prob)
    return _sc_gather_combine(model_prob, tgt, s2d.reshape(_N_TOKENS),
                              z2d.reshape(_N_TOKENS))


# speedup vs baseline: 1.0504x; 1.0504x over previous
import functools
import math

import jax
import jax.numpy as jnp
from jax import lax
from jax.experimental import pallas as pl
from jax.experimental.pallas import tpu as pltpu
from jax.experimental.pallas import tpu_sc as plsc

_SMOOTHING = 0.1
_VOCAB = 32000
_N_TOKENS = 2048
_CONF = 1.0 - _SMOOTHING
_SVAL = _SMOOTHING / float(_VOCAB - 2)
_C_HIT = (_VOCAB - 2) * _SVAL * math.log(_SVAL) + _CONF * math.log(_CONF)
_C_IGN = (_VOCAB - 1) * _SVAL * math.log(_SVAL) + _CONF * math.log(_CONF)

_BC = 3200
_NBLK = _VOCAB // _BC

_LN = 16
_NWORKERS = 32
_TOK_PER_W = _N_TOKENS // _NWORKERS
_NCHUNK = _TOK_PER_W // _LN


def _sc_body(table_hbm, tgt_hbm, s_hbm, out_hbm, tgt_v, s_v, row_v,
             lane_v, zrows_v, trows_v, out_v, semz, semt):
    wid = lax.axis_index("s") * 2 + lax.axis_index("c")
    base = wid * _TOK_PER_W
    pltpu.sync_copy(tgt_hbm.at[pl.ds(base, _TOK_PER_W)], tgt_v)
    pltpu.sync_copy(s_hbm.at[pl.ds(base, _TOK_PER_W)], s_v)
    for k in range(_NCHUNK):
        t16 = tgt_v[pl.ds(k * _LN, _LN)]
        i16 = base + k * _LN + lax.iota(jnp.int32, _LN)
        row_v[pl.ds(k * _LN, _LN)] = i16
        lane_v[pl.ds(k * _LN, _LN)] = t16 & 127
    zcopy = pltpu.make_async_copy(
        table_hbm.at[row_v, pl.ds(0, 128)], zrows_v, semz)
    zcopy.start()
    tcopies = []
    lane_iota = lax.iota(jnp.int32, _LN)
    for j in range(_TOK_PER_W):
        t16 = tgt_v[pl.ds((j // _LN) * _LN, _LN)]
        tj = jnp.sum(jnp.where(lane_iota == (j % _LN), t16, 0))
        c128 = pl.multiple_of(tj - (tj & 127), 128)
        r8 = pl.multiple_of(base + (j // 8) * 8, 8)
        cp = pltpu.make_async_copy(
            table_hbm.at[pl.ds(r8, 8), pl.ds(c128, 128)],
            trows_v.at[j], semt)
        cp.start()
        tcopies.append(cp)
    zcopy.wait()
    for cp in tcopies:
        cp.wait()
    zero16 = lane_iota & 0
    for k in range(_NCHUNK):
        loc = k * _LN + lane_iota
        sub = loc & 7
        ln = lane_v[pl.ds(k * _LN, _LN)]
        tg = tgt_v[pl.ds(k * _LN, _LN)]
        sv = s_v[pl.ds(k * _LN, _LN)]
        t = plsc.load_gather(trows_v, [loc, sub, ln])
        z = plsc.load_gather(zrows_v, [loc, zero16])
        hit_val = _C_HIT + _SVAL * z + (_SVAL - _CONF) * t
        ign_val = _C_IGN + (_SVAL - _CONF) * z
        out_v[pl.ds(k * _LN, _LN)] = jnp.where(
            tg == 0, ign_val, hit_val) - _SVAL * sv
    pltpu.sync_copy(out_v, out_hbm.at[pl.ds(base, _TOK_PER_W)])


def _sc_gather_combine(table, tgt, s1d):
    mesh = plsc.VectorSubcoreMesh(core_axis_name="c", subcore_axis_name="s")
    f = pl.kernel(
        _sc_body,
        out_type=jax.ShapeDtypeStruct((_N_TOKENS,), jnp.float32),
        mesh=mesh,
        scratch_types=[
            pltpu.VMEM((_TOK_PER_W,), jnp.int32),
            pltpu.VMEM((_TOK_PER_W,), jnp.float32),
            pltpu.VMEM((_TOK_PER_W,), jnp.int32),
            pltpu.VMEM((_TOK_PER_W,), jnp.int32),
            pltpu.VMEM((_TOK_PER_W, 128), jnp.float32),
            pltpu.VMEM((_TOK_PER_W, 8, 128), jnp.float32),
            pltpu.VMEM((_TOK_PER_W,), jnp.float32),
            pltpu.SemaphoreType.DMA,
            pltpu.SemaphoreType.DMA,
        ],
        compiler_params=pltpu.CompilerParams(needs_layout_passes=False),
    )
    return f(table, tgt, s1d)


_BR = 128
_NRB = _N_TOKENS // _BR


def _tc_rowsum_body(x_ref, s_ref):
    s_ref[...] = jnp.sum(x_ref[...], axis=1, keepdims=True)


def _tc_rowsum(x, interpret=False):
    return pl.pallas_call(
        _tc_rowsum_body,
        grid=(_NRB,),
        in_specs=[pl.BlockSpec((_BR, _VOCAB), lambda j: (j, 0))],
        out_specs=pl.BlockSpec((_BR, 1), lambda j: (j, 0)),
        out_shape=jax.ShapeDtypeStruct((_N_TOKENS, 1), jnp.float32),
        interpret=interpret,
    )(x)


def kernel(model_prob, target):
    tgt = target.astype(jnp.int32)
    s1d = _tc_rowsum(model_prob).reshape(_N_TOKENS)
    return _sc_gather_combine(model_prob, tgt, s1d)
